# baseline (speedup 1.0000x reference)
import jax
import jax.numpy as jnp
from jax import lax
from jax.experimental import pallas as pl
from jax.experimental.pallas import tpu as pltpu
from jax.experimental.pallas import tpu_sc as plsc

M = 32
K = 8192
D = 64
N = 16384

NC = 2
NS = 16
MPC = M // NC
MG = 8
DG = 8
NCH = 2048
NCHN = N // NCH
CBT = NCH // 128
L = 16
MT = MPC // NC


def _lookup_body(codes4, cb5, out4, stage_v, tab_v, idx_v, out_v, codes_sh,
                 sem_st, sem_tab, sem_idx, sem_out):
    c = lax.axis_index("c")
    s = lax.axis_index("s")
    mg = s // DG
    dg = s % DG
    mrow0 = MPC * c

    def tab_copy(mi):
        m = mrow0 + mg * MG + mi
        return pltpu.make_async_copy(cb5.at[m, dg], tab_v, sem_tab)

    tab_copy(0).start()

    for r in range(2):
        u = 2 * s + r
        st_mg = u // 16
        st_nb = u % 16
        mb = 2 * c + st_mg

        def st_copy():
            return pltpu.make_async_copy(
                codes4.at[mb, pl.ds(st_nb * 8, 8)], stage_v, sem_st)

        st_copy().start()
        st_copy().wait()
        for mr in range(MG):
            pltpu.async_copy(
                stage_v.at[:, mr],
                codes_sh.at[st_mg * MG + mr, pl.ds(st_nb * 8, 8)],
                sem_st)
        for mr in range(MG):
            pltpu.make_async_copy(
                stage_v.at[:, mr],
                codes_sh.at[st_mg * MG + mr, pl.ds(st_nb * 8, 8)],
                sem_st).wait()
    plsc.subcore_barrier()

    def idx_copy(mi, ch, p):
        return pltpu.make_async_copy(
            codes_sh.at[mg * MG + mi, pl.ds(ch * CBT, CBT)],
            idx_v.at[p], sem_idx.at[p])

    def out_copy(mi, ch, p):
        m = mrow0 + mg * MG + mi
        rb = m * (D // 8) + dg
        return pltpu.make_async_copy(
            out_v.at[p],
            out4.at[rb, pl.ds(ch * CBT, CBT)],
            sem_out.at[p])

    idx_copy(0, 0, 0).start()

    dvecs = [jnp.full((L,), d, jnp.int32) for d in range(DG)]
    kmask = jnp.full((L,), 127, jnp.int32)

    def idx_wait(p):
        idx_copy(0, 0, p).wait()

    def out_wait(p):
        out_copy(0, 0, p).wait()

    def chunk(mi, ch, p, first, idx_pf, idx_pf_next_m):
        idx_wait(p)
        idx_pf(p)
        idx_pf_next_m(p)

        @pl.when(jnp.logical_not(first))
        def _():
            out_wait(p)

        def gcb(cb):
            for g in range(128 // L):
                iv = idx_v[p, cb, pl.ds(g * L, L)]
                hi = lax.shift_right_logical(iv, 7)
                lo = lax.bitwise_and(iv, kmask)
                for d in range(DG):
                    vals = plsc.load_gather(tab_v, [hi, dvecs[d], lo])
                    out_v[p, cb, d, pl.ds(g * L, L)] = vals

        pl.loop(0, CBT)(gcb)
        out_copy(mi, ch, p).start()

    def m_step(mi):
        tab_copy(0).wait()

        def ch_step(chv):
            def pf_a(p):
                idx_copy(mi, chv + 1, 1 - p).start()

            def pf_none(p):
                pass

            def pf_b(p):
                @pl.when(chv + 2 < NCHN)
                def _():
                    idx_copy(mi, chv + 2, 1 - p).start()

            def pf_b2(p):
                @pl.when(jnp.logical_and(chv + 2 >= NCHN, mi + 1 < MT))
                def _():
                    idx_copy(mi + 1, 0, 1 - p).start()

            first0 = jnp.logical_and(mi == 0, chv == 0)
            chunk(mi, chv, 0, first0, pf_a, pf_none)
            chunk(mi, chv + 1, 1, first0, pf_b, pf_b2)

        pl.loop(0, NCHN, step=2)(ch_step)

        @pl.when(mi + 1 < MT)
        def _():
            tab_copy(mi + 1).start()

    pl.loop(0, MT)(m_step)

    out_wait(0)
    out_wait(1)


@jax.jit
def kernel(codes, codebook):
    codes4 = (codes.astype(jnp.int32).T
              .reshape(4, 8, 128, 128)
              .transpose(0, 2, 1, 3))
    cb5 = (codebook.transpose(0, 2, 1)
           .reshape(M, 8, 8, 64, 128)
           .transpose(0, 1, 3, 2, 4))

    mesh = plsc.VectorSubcoreMesh(core_axis_name="c", subcore_axis_name="s")
    out4 = pl.kernel(
        _lookup_body,
        mesh=mesh,
        compiler_params=pltpu.CompilerParams(use_tc_tiling_on_sc=True,
                                             needs_layout_passes=False),
        out_type=jax.ShapeDtypeStruct((M * D // 8, N // 128, 8, 128),
                                      jnp.float32),
        scratch_types=[
            pltpu.VMEM((8, MG, 128), jnp.int32),
            pltpu.VMEM((K // 128, DG, 128), jnp.float32),
            pltpu.VMEM((2, CBT, 128), jnp.int32),
            pltpu.VMEM((2, CBT, DG, 128), jnp.float32),
            pltpu.VMEM_SHARED((MPC, N // 128, 128), jnp.int32),
            pltpu.SemaphoreType.DMA,
            pltpu.SemaphoreType.DMA,
            pltpu.SemaphoreType.DMA((2,)),
            pltpu.SemaphoreType.DMA((2,)),
        ],
    )(codes4, cb5)
    return (out4.transpose(0, 2, 1, 3)
            .reshape(M, D, N)
            .transpose(2, 0, 1))


# speedup vs baseline: 2.1553x; 2.1553x over previous
import jax
import jax.numpy as jnp
from jax import lax
from jax.experimental import pallas as pl
from jax.experimental.pallas import tpu as pltpu
from jax.experimental.pallas import tpu_sc as plsc

M = 32
K = 8192
D = 64
N = 16384

NC = 2
NS = 16
MPC = M // NC
MG = 8
DG = 8
NCH = 2048
NCHN = N // NCH
CBT = NCH // 128
L = 16
MT = MPC // NC


def _lookup_body(codes4, cb5, out4, stage_v, tab_v, idx_v, out_v, codes_sh,
                 sem_st, sem_tab, sem_idx, sem_out):
    c = lax.axis_index("c")
    s = lax.axis_index("s")
    mg = s // DG
    dg = s % DG
    mrow0 = MPC * c

    def tab_copy(mi):
        m = mrow0 + mg * MG + mi
        return pltpu.make_async_copy(cb5.at[m, dg], tab_v, sem_tab)

    tab_copy(0).start()

    for r in range(2):
        u = 2 * s + r
        st_mg = u // 16
        st_nb = u % 16
        mb = 2 * c + st_mg

        def st_copy():
            return pltpu.make_async_copy(
                codes4.at[mb, pl.ds(st_nb * 8, 8)], stage_v, sem_st)

        st_copy().start()
        st_copy().wait()
        for mr in range(MG):
            pltpu.async_copy(
                stage_v.at[:, mr],
                codes_sh.at[st_mg * MG + mr, pl.ds(st_nb * 8, 8)],
                sem_st)
        for mr in range(MG):
            pltpu.make_async_copy(
                stage_v.at[:, mr],
                codes_sh.at[st_mg * MG + mr, pl.ds(st_nb * 8, 8)],
                sem_st).wait()
    plsc.subcore_barrier()

    def idx_copy(mi, ch, p):
        return pltpu.make_async_copy(
            codes_sh.at[mg * MG + mi, pl.ds(ch * CBT, CBT)],
            idx_v.at[p], sem_idx.at[p])

    def out_copy(mi, ch, p):
        m = mrow0 + mg * MG + mi
        rb = m * (D // 8) + dg
        return pltpu.make_async_copy(
            out_v.at[p],
            out4.at[rb, pl.ds(ch * CBT, CBT)],
            sem_out.at[p])

    idx_copy(0, 0, 0).start()

    dvecs = [jnp.full((L,), d, jnp.int32) for d in range(DG)]
    kmask = jnp.full((L,), 127, jnp.int32)

    def idx_wait(p):
        idx_copy(0, 0, p).wait()

    def out_wait(p):
        out_copy(0, 0, p).wait()

    def chunk(mi, ch, p, first, idx_pf, idx_pf_next_m):
        idx_wait(p)
        idx_pf(p)
        idx_pf_next_m(p)

        @pl.when(jnp.logical_not(first))
        def _():
            out_wait(p)

        def gcb(cb):
            for g0 in range(0, 128 // L, 2):
                parts = []
                for g in (g0, g0 + 1):
                    iv = idx_v[p, cb, pl.ds(g * L, L)]
                    hi = lax.shift_right_logical(iv, 7)
                    lo = lax.bitwise_and(iv, kmask)
                    parts.append([plsc.load_gather(tab_v, [hi, dvecs[d], lo])
                                  for d in range(DG)])
                for gi, g in enumerate((g0, g0 + 1)):
                    for d in range(DG):
                        out_v[p, cb, d, pl.ds(g * L, L)] = parts[gi][d]

        pl.loop(0, CBT)(gcb)
        out_copy(mi, ch, p).start()

    def m_step(mi):
        tab_copy(0).wait()

        def ch_step(chv):
            def pf_a(p):
                idx_copy(mi, chv + 1, 1 - p).start()

            def pf_none(p):
                pass

            def pf_b(p):
                @pl.when(chv + 2 < NCHN)
                def _():
                    idx_copy(mi, chv + 2, 1 - p).start()

            def pf_b2(p):
                @pl.when(jnp.logical_and(chv + 2 >= NCHN, mi + 1 < MT))
                def _():
                    idx_copy(mi + 1, 0, 1 - p).start()

            first0 = jnp.logical_and(mi == 0, chv == 0)
            chunk(mi, chv, 0, first0, pf_a, pf_none)
            chunk(mi, chv + 1, 1, first0, pf_b, pf_b2)

        pl.loop(0, NCHN, step=2)(ch_step)

        @pl.when(mi + 1 < MT)
        def _():
            tab_copy(mi + 1).start()

    pl.loop(0, MT)(m_step)

    out_wait(0)
    out_wait(1)


@jax.jit
def kernel(codes, codebook):
    codes4 = (codes.astype(jnp.int32).T
              .reshape(4, 8, 128, 128)
              .transpose(0, 2, 1, 3))
    cb5 = (codebook.transpose(0, 2, 1)
           .reshape(M, 8, 8, 64, 128)
           .transpose(0, 1, 3, 2, 4))

    mesh = plsc.VectorSubcoreMesh(core_axis_name="c", subcore_axis_name="s")
    out4 = pl.kernel(
        _lookup_body,
        mesh=mesh,
        compiler_params=pltpu.CompilerParams(use_tc_tiling_on_sc=True,
                                             needs_layout_passes=False),
        out_type=jax.ShapeDtypeStruct((M * D // 8, N // 128, 8, 128),
                                      jnp.float32),
        scratch_types=[
            pltpu.VMEM((8, MG, 128), jnp.int32),
            pltpu.VMEM((K // 128, DG, 128), jnp.float32),
            pltpu.VMEM((2, CBT, 128), jnp.int32),
            pltpu.VMEM((2, CBT, DG, 128), jnp.float32),
            pltpu.VMEM_SHARED((MPC, N // 128, 128), jnp.int32),
            pltpu.SemaphoreType.DMA,
            pltpu.SemaphoreType.DMA,
            pltpu.SemaphoreType.DMA((2,)),
            pltpu.SemaphoreType.DMA((2,)),
        ],
    )(codes4, cb5)
    return (out4.transpose(0, 2, 1, 3)
            .reshape(M, D, N)
            .transpose(2, 0, 1))


# speedup vs baseline: 2.2325x; 1.0358x over previous
import jax
import jax.numpy as jnp
from jax import lax
from jax.experimental import pallas as pl
from jax.experimental.pallas import tpu as pltpu
from jax.experimental.pallas import tpu_sc as plsc

M = 32
K = 8192
D = 64
N = 16384

NC = 2
NS = 16
MPC = M // NC
MG = 8
DG = 8
NCH = 2048
NCHN = N // NCH
CBT = NCH // 128
L = 16
MT = MPC // NC


def _lookup_body(codes4, cb5, out4, stage_v, tab_v, idx_v, out_v, codes_sh,
                 sem_st, sem_tab, sem_idx, sem_out):
    c = lax.axis_index("c")
    s = lax.axis_index("s")
    mg = s // DG
    dg = s % DG
    mrow0 = MPC * c

    def tab_copy(mi):
        m = mrow0 + mg * MG + mi
        return pltpu.make_async_copy(cb5.at[m, dg], tab_v, sem_tab)

    tab_copy(0).start()

    for r in range(2):
        u = 2 * s + r
        st_mg = u // 16
        st_nb = u % 16
        mb = 2 * c + st_mg

        def st_copy():
            return pltpu.make_async_copy(
                codes4.at[mb, pl.ds(st_nb * 8, 8)], stage_v, sem_st)

        st_copy().start()
        st_copy().wait()
        for mr in range(MG):
            pltpu.async_copy(
                stage_v.at[:, mr],
                codes_sh.at[st_mg * MG + mr, pl.ds(st_nb * 8, 8)],
                sem_st)
        for mr in range(MG):
            pltpu.make_async_copy(
                stage_v.at[:, mr],
                codes_sh.at[st_mg * MG + mr, pl.ds(st_nb * 8, 8)],
                sem_st).wait()
    plsc.subcore_barrier()

    def idx_copy(mi, ch, p):
        return pltpu.make_async_copy(
            codes_sh.at[mg * MG + mi, pl.ds(ch * CBT, CBT)],
            idx_v.at[p], sem_idx.at[p])

    def out_copy(mi, ch, p):
        m = mrow0 + mg * MG + mi
        rb = m * (D // 8) + dg
        return pltpu.make_async_copy(
            out_v.at[p],
            out4.at[rb, pl.ds(ch * CBT, CBT)],
            sem_out.at[p])

    idx_copy(0, 0, 0).start()

    dvecs = [jnp.full((L,), d, jnp.int32) for d in range(DG)]
    kmask = jnp.full((L,), 127, jnp.int32)

    def idx_wait(p):
        idx_copy(0, 0, p).wait()

    def out_wait(p):
        out_copy(0, 0, p).wait()

    def chunk(mi, ch, p, first, idx_pf, idx_pf_next_m):
        idx_wait(p)
        idx_pf(p)
        idx_pf_next_m(p)

        @pl.when(jnp.logical_not(first))
        def _():
            out_wait(p)

        def gcb(cb):
            def gather_round(g0):
                parts = []
                for g in (g0, g0 + 1):
                    iv = idx_v[p, cb, pl.ds(g * L, L)]
                    hi = lax.shift_right_logical(iv, 7)
                    lo = lax.bitwise_and(iv, kmask)
                    parts.append([plsc.load_gather(tab_v, [hi, dvecs[d], lo])
                                  for d in range(DG)])
                return parts

            def store_round(g0, parts):
                for gi, g in enumerate((g0, g0 + 1)):
                    for d in range(DG):
                        out_v[p, cb, d, pl.ds(g * L, L)] = parts[gi][d]

            prev = gather_round(0)
            for g0 in range(2, 128 // L, 2):
                cur = gather_round(g0)
                store_round(g0 - 2, prev)
                prev = cur
            store_round(128 // L - 2, prev)

        pl.loop(0, CBT)(gcb)
        out_copy(mi, ch, p).start()

    def m_step(mi):
        tab_copy(0).wait()

        def ch_step(chv):
            def pf_a(p):
                idx_copy(mi, chv + 1, 1 - p).start()

            def pf_none(p):
                pass

            def pf_b(p):
                @pl.when(chv + 2 < NCHN)
                def _():
                    idx_copy(mi, chv + 2, 1 - p).start()

            def pf_b2(p):
                @pl.when(jnp.logical_and(chv + 2 >= NCHN, mi + 1 < MT))
                def _():
                    idx_copy(mi + 1, 0, 1 - p).start()

            first0 = jnp.logical_and(mi == 0, chv == 0)
            chunk(mi, chv, 0, first0, pf_a, pf_none)
            chunk(mi, chv + 1, 1, first0, pf_b, pf_b2)

        pl.loop(0, NCHN, step=2)(ch_step)

        @pl.when(mi + 1 < MT)
        def _():
            tab_copy(mi + 1).start()

    pl.loop(0, MT)(m_step)

    out_wait(0)
    out_wait(1)


@jax.jit
def kernel(codes, codebook):
    codes4 = (codes.astype(jnp.int32).T
              .reshape(4, 8, 128, 128)
              .transpose(0, 2, 1, 3))
    cb5 = (codebook.transpose(0, 2, 1)
           .reshape(M, 8, 8, 64, 128)
           .transpose(0, 1, 3, 2, 4))

    mesh = plsc.VectorSubcoreMesh(core_axis_name="c", subcore_axis_name="s")
    out4 = pl.kernel(
        _lookup_body,
        mesh=mesh,
        compiler_params=pltpu.CompilerParams(use_tc_tiling_on_sc=True,
                                             needs_layout_passes=False),
        out_type=jax.ShapeDtypeStruct((M * D // 8, N // 128, 8, 128),
                                      jnp.float32),
        scratch_types=[
            pltpu.VMEM((8, MG, 128), jnp.int32),
            pltpu.VMEM((K // 128, DG, 128), jnp.float32),
            pltpu.VMEM((2, CBT, 128), jnp.int32),
            pltpu.VMEM((2, CBT, DG, 128), jnp.float32),
            pltpu.VMEM_SHARED((MPC, N // 128, 128), jnp.int32),
            pltpu.SemaphoreType.DMA,
            pltpu.SemaphoreType.DMA,
            pltpu.SemaphoreType.DMA((2,)),
            pltpu.SemaphoreType.DMA((2,)),
        ],
    )(codes4, cb5)
    return (out4.transpose(0, 2, 1, 3)
            .reshape(M, D, N)
            .transpose(2, 0, 1))
